# baseline (speedup 1.0000x reference)
import functools

import jax
import jax.numpy as jnp
from jax import lax
from jax.experimental import pallas as pl
from jax.experimental.pallas import tpu as pltpu
from jax.experimental.pallas import tpu_sc as plsc

V = 100000
BASE = 32
DIMS = (32, 32, 32, 51, 102)
_B, _L = 16384, 50


_CB = 1024
_GRID = (V + _CB - 1) // _CB


def _build_body(ba_ref, e0_ref, e1_ref, e2_ref, e3_ref, e4_ref,
                w3_ref, b3_ref, w4_ref, b4_ref, out_ref):
    ba = ba_ref[...]
    p3 = jnp.dot(w3_ref[...], e3_ref[...],
                 preferred_element_type=jnp.float32) + b3_ref[...].T
    p4 = jnp.dot(w4_ref[...], e4_ref[...],
                 preferred_element_type=jnp.float32) + b4_ref[...].T
    ct = jnp.where(ba == 0, e0_ref[...], 0.0)
    ct = ct + jnp.where(ba == 1, e1_ref[...], 0.0)
    ct = ct + jnp.where(ba == 2, e2_ref[...], 0.0)
    ct = ct + jnp.where(ba == 3, p3, 0.0)
    ct = ct + jnp.where(ba == 4, p4, 0.0)
    out_ref[:, :BASE] = ct.T


def _build_combined(ba, e0t, e1t, e2t, e3t, e4t, W3, b3, W4, b4):
    ba2 = ba.reshape(1, V).astype(jnp.int32)
    b3r = b3.reshape(1, BASE)
    b4r = b4.reshape(1, BASE)
    col = lambda i: (0, i)
    fixed = lambda i: (0, 0)
    return pl.pallas_call(
        _build_body,
        grid=(_GRID,),
        in_specs=[
            pl.BlockSpec((1, _CB), col),
            pl.BlockSpec((DIMS[0], _CB), col),
            pl.BlockSpec((DIMS[1], _CB), col),
            pl.BlockSpec((DIMS[2], _CB), col),
            pl.BlockSpec((DIMS[3], _CB), col),
            pl.BlockSpec((DIMS[4], _CB), col),
            pl.BlockSpec((BASE, DIMS[3]), fixed),
            pl.BlockSpec((1, BASE), fixed),
            pl.BlockSpec((BASE, DIMS[4]), fixed),
            pl.BlockSpec((1, BASE), fixed),
        ],
        out_specs=pl.BlockSpec((_CB, 128), lambda i: (i, 0)),
        out_shape=jax.ShapeDtypeStruct((V, 128), jnp.float32),
    )(ba2, e0t, e1t, e2t, e3t, e4t, W3, b3r, W4, b4r)


_NC = 2
_NS = 16
_NW = _NC * _NS
_NTOK = _B * _L
_PER_W = _NTOK // _NW
_CH = 1280
_NCHUNK = _PER_W // _CH


@functools.partial(
    pl.kernel,
    mesh=plsc.VectorSubcoreMesh(core_axis_name="c", subcore_axis_name="s",
                                num_cores=_NC),
    out_type=jax.ShapeDtypeStruct((_NTOK, BASE), jnp.float32),
    scratch_types=[
        pltpu.VMEM((_CH,), jnp.int32),
        pltpu.VMEM((_CH,), jnp.int32),
        pltpu.VMEM((_CH,), jnp.int32),
        pltpu.VMEM((_CH,), jnp.int32),
        pltpu.VMEM((_CH, BASE), jnp.float32),
        pltpu.VMEM((_CH, BASE), jnp.float32),
        pltpu.SemaphoreType.DMA,
        pltpu.SemaphoreType.DMA,
        pltpu.SemaphoreType.DMA,
        pltpu.SemaphoreType.DMA,
    ],
    compiler_params=pltpu.CompilerParams(use_tc_tiling_on_sc=False),
)
def _sc_gather(c_hbm, idx_hbm, dst_hbm, out_hbm,
               idx0, idx1, dst0, dst1, rows0, rows1, gs0, gs1, ss0, ss1):
    wid = lax.axis_index("s") * _NC + lax.axis_index("c")
    base = wid * _PER_W
    idxb, dstb = (idx0, idx1), (dst0, dst1)
    rows, gs, ss = (rows0, rows1), (gs0, gs1), (ss0, ss1)

    def load(c, slot):
        off = base + c * _CH
        pltpu.sync_copy(idx_hbm.at[pl.ds(off, _CH)], idxb[slot])
        pltpu.sync_copy(dst_hbm.at[pl.ds(off, _CH)], dstb[slot])

    load(0, 0)
    g_h = [pltpu.async_copy(c_hbm.at[idxb[0]], rows[0], gs[0]), None]
    s_h = [None, None]
    for c in range(_NCHUNK):
        cur, nxt = c % 2, (c + 1) % 2
        if c + 1 < _NCHUNK:
            if s_h[nxt] is not None:
                s_h[nxt].wait()
            load(c + 1, nxt)
            g_h[nxt] = pltpu.async_copy(c_hbm.at[idxb[nxt]], rows[nxt], gs[nxt])
        g_h[cur].wait()
        s_h[cur] = pltpu.async_copy(rows[cur], out_hbm.at[dstb[cur]], ss[cur])
    for slot in (0, 1):
        if s_h[slot] is not None:
            s_h[slot].wait()


_BC = 2048
_NBC = _B // _BC


def _tr_body(g_ref, out_ref):
    t = g_ref[...].T.reshape(4, BASE, _BC // 4)
    out_ref[...] = jnp.transpose(t, (1, 0, 2)).reshape(1, BASE, _BC)


def _transpose_out(g128):
    return pl.pallas_call(
        _tr_body,
        grid=(_L, _NBC),
        in_specs=[
            pl.BlockSpec((_BC // 4, 128), lambda l, b: (l * _NBC + b, 0)),
        ],
        out_specs=pl.BlockSpec((1, BASE, _BC), lambda l, b: (l, 0, b)),
        out_shape=jax.ShapeDtypeStruct((_L, BASE, _B), jnp.float32),
    )(g128)


def kernel(x, bucket_assignment, emb0, emb1, emb2, emb3, emb4, W3, b3, W4, b4):
    c_pad = _build_combined(bucket_assignment, emb0.T, emb1.T, emb2.T,
                            emb3.T, emb4.T, W3, b3, W4, b4)
    c_rows = c_pad.reshape(4 * V, BASE)
    idx = x.T.reshape(-1).astype(jnp.int32) * 4
    p = jnp.arange(_NTOK, dtype=jnp.int32)
    u, l_ = p % (_B), p // _B
    bc, v_ = u // _BC, u % _BC
    q, r = v_ // (_BC // 4), v_ % (_BC // 4)
    dst = ((l_ * _NBC + bc) * (_BC // 4) + r) * 4 + q
    g = _sc_gather(c_rows, idx, dst)
    g128 = g.reshape(_NTOK // 4, 128)
    out_phys = _transpose_out(g128)
    return jnp.transpose(out_phys, (2, 0, 1))


# speedup vs baseline: 1.3986x; 1.3986x over previous
import functools

import jax
import jax.numpy as jnp
from jax import lax
from jax.experimental import pallas as pl
from jax.experimental.pallas import tpu as pltpu
from jax.experimental.pallas import tpu_sc as plsc

V = 100000
BASE = 32
DIMS = (32, 32, 32, 51, 102)
_B, _L = 16384, 50


_CB = 2048
_GRID = (V + _CB - 1) // _CB


def _build_body(ba_ref, e0_ref, e1_ref, e2_ref, e3_ref, e4_ref,
                w3_ref, b3_ref, w4_ref, b4_ref, out_ref):
    ba = ba_ref[...]
    p3 = jnp.dot(w3_ref[...], e3_ref[...],
                 preferred_element_type=jnp.float32) + b3_ref[...].T
    p4 = jnp.dot(w4_ref[...], e4_ref[...],
                 preferred_element_type=jnp.float32) + b4_ref[...].T
    ct = jnp.where(ba == 0, e0_ref[...], 0.0)
    ct = ct + jnp.where(ba == 1, e1_ref[...], 0.0)
    ct = ct + jnp.where(ba == 2, e2_ref[...], 0.0)
    ct = ct + jnp.where(ba == 3, p3, 0.0)
    ct = ct + jnp.where(ba == 4, p4, 0.0)
    out_ref[:, :BASE] = ct.T


def _build_combined(ba, e0t, e1t, e2t, e3t, e4t, W3, b3, W4, b4):
    ba2 = ba.reshape(1, V).astype(jnp.int32)
    b3r = b3.reshape(1, BASE)
    b4r = b4.reshape(1, BASE)
    col = lambda i: (0, i)
    fixed = lambda i: (0, 0)
    return pl.pallas_call(
        _build_body,
        grid=(_GRID,),
        in_specs=[
            pl.BlockSpec((1, _CB), col),
            pl.BlockSpec((DIMS[0], _CB), col),
            pl.BlockSpec((DIMS[1], _CB), col),
            pl.BlockSpec((DIMS[2], _CB), col),
            pl.BlockSpec((DIMS[3], _CB), col),
            pl.BlockSpec((DIMS[4], _CB), col),
            pl.BlockSpec((BASE, DIMS[3]), fixed),
            pl.BlockSpec((1, BASE), fixed),
            pl.BlockSpec((BASE, DIMS[4]), fixed),
            pl.BlockSpec((1, BASE), fixed),
        ],
        out_specs=pl.BlockSpec((_CB, 128), lambda i: (i, 0)),
        out_shape=jax.ShapeDtypeStruct((V, 128), jnp.float32),
    )(ba2, e0t, e1t, e2t, e3t, e4t, W3, b3r, W4, b4r)


_NC = 2
_NS = 16
_NW = _NC * _NS
_NTOK = _B * _L
_PER_W = _NTOK // _NW
_CH = 2560
_NCHUNK = _PER_W // _CH


@functools.partial(
    pl.kernel,
    mesh=plsc.VectorSubcoreMesh(core_axis_name="c", subcore_axis_name="s",
                                num_cores=_NC),
    out_type=jax.ShapeDtypeStruct((_NTOK, BASE), jnp.float32),
    scratch_types=[
        pltpu.VMEM((_CH,), jnp.int32),
        pltpu.VMEM((_CH,), jnp.int32),
        pltpu.VMEM((_CH, BASE), jnp.float32),
        pltpu.SemaphoreType.DMA,
        pltpu.SemaphoreType.DMA,
    ],
    compiler_params=pltpu.CompilerParams(use_tc_tiling_on_sc=False),
)
def _sc_gather(c_hbm, idx_hbm, dst_hbm, out_hbm, idxc, dstc, rows, sem, sem2):
    wid = lax.axis_index("s") * _NC + lax.axis_index("c")
    base = wid * _PER_W
    for c in range(_NCHUNK):
        off = base + c * _CH
        pltpu.sync_copy(idx_hbm.at[pl.ds(off, _CH)], idxc)
        pltpu.sync_copy(dst_hbm.at[pl.ds(off, _CH)], dstc)
        pltpu.async_copy(c_hbm.at[idxc], rows, sem).wait()
        pltpu.async_copy(rows, out_hbm.at[dstc], sem2).wait()


_BC = 4096
_NBC = _B // _BC


def _tr_body(g_ref, out_ref):
    t = g_ref[...].T.reshape(4, BASE, _BC // 4)
    out_ref[...] = jnp.transpose(t, (1, 0, 2)).reshape(1, BASE, _BC)


def _transpose_out(g128):
    return pl.pallas_call(
        _tr_body,
        grid=(_L, _NBC),
        in_specs=[
            pl.BlockSpec((_BC // 4, 128), lambda l, b: (l * _NBC + b, 0)),
        ],
        out_specs=pl.BlockSpec((1, BASE, _BC), lambda l, b: (l, 0, b)),
        out_shape=jax.ShapeDtypeStruct((_L, BASE, _B), jnp.float32),
    )(g128)


def kernel(x, bucket_assignment, emb0, emb1, emb2, emb3, emb4, W3, b3, W4, b4):
    c_pad = _build_combined(bucket_assignment, emb0.T, emb1.T, emb2.T,
                            emb3.T, emb4.T, W3, b3, W4, b4)
    c_rows = c_pad.reshape(4 * V, BASE)
    idx = x.T.reshape(-1).astype(jnp.int32) * 4
    p = jnp.arange(_NTOK, dtype=jnp.int32)
    u, l_ = p % (_B), p // _B
    bc, v_ = u // _BC, u % _BC
    q, r = v_ // (_BC // 4), v_ % (_BC // 4)
    dst = ((l_ * _NBC + bc) * (_BC // 4) + r) * 4 + q
    g = _sc_gather(c_rows, idx, dst)
    g128 = g.reshape(_NTOK // 4, 128)
    out_phys = _transpose_out(g128)
    return jnp.transpose(out_phys, (2, 0, 1))


# speedup vs baseline: 1.6919x; 1.2097x over previous
import functools

import jax
import jax.numpy as jnp
from jax import lax
from jax.experimental import pallas as pl
from jax.experimental.pallas import tpu as pltpu
from jax.experimental.pallas import tpu_sc as plsc

V = 100000
BASE = 32
DIMS = (32, 32, 32, 51, 102)
_B, _L = 16384, 50


_CB = 4096
_GRID = (V + _CB - 1) // _CB


def _build_body(ba_ref, e0_ref, e1_ref, e2_ref, e3_ref, e4_ref,
                w3_ref, b3_ref, w4_ref, b4_ref, out_ref):
    ba = ba_ref[...]
    p3 = jnp.dot(w3_ref[...], e3_ref[...],
                 preferred_element_type=jnp.float32) + b3_ref[...].T
    p4 = jnp.dot(w4_ref[...], e4_ref[...],
                 preferred_element_type=jnp.float32) + b4_ref[...].T
    ct = jnp.where(ba == 0, e0_ref[...], 0.0)
    ct = ct + jnp.where(ba == 1, e1_ref[...], 0.0)
    ct = ct + jnp.where(ba == 2, e2_ref[...], 0.0)
    ct = ct + jnp.where(ba == 3, p3, 0.0)
    ct = ct + jnp.where(ba == 4, p4, 0.0)
    out_ref[:, :BASE] = ct.T


def _build_combined(ba, e0t, e1t, e2t, e3t, e4t, W3, b3, W4, b4):
    ba2 = ba.reshape(1, V).astype(jnp.int32)
    b3r = b3.reshape(1, BASE)
    b4r = b4.reshape(1, BASE)
    col = lambda i: (0, i)
    fixed = lambda i: (0, 0)
    return pl.pallas_call(
        _build_body,
        grid=(_GRID,),
        in_specs=[
            pl.BlockSpec((1, _CB), col),
            pl.BlockSpec((DIMS[0], _CB), col),
            pl.BlockSpec((DIMS[1], _CB), col),
            pl.BlockSpec((DIMS[2], _CB), col),
            pl.BlockSpec((DIMS[3], _CB), col),
            pl.BlockSpec((DIMS[4], _CB), col),
            pl.BlockSpec((BASE, DIMS[3]), fixed),
            pl.BlockSpec((1, BASE), fixed),
            pl.BlockSpec((BASE, DIMS[4]), fixed),
            pl.BlockSpec((1, BASE), fixed),
        ],
        out_specs=pl.BlockSpec((_CB, 128), lambda i: (i, 0)),
        out_shape=jax.ShapeDtypeStruct((V, 128), jnp.float32),
    )(ba2, e0t, e1t, e2t, e3t, e4t, W3, b3r, W4, b4r)


_NC = 2
_NS = 16
_NW = _NC * _NS
_NTOK = _B * _L
_PER_W = _NTOK // _NW
_CH = 2560
_NCHUNK = _PER_W // _CH


@functools.partial(
    pl.kernel,
    mesh=plsc.VectorSubcoreMesh(core_axis_name="c", subcore_axis_name="s",
                                num_cores=_NC),
    out_type=jax.ShapeDtypeStruct((_NTOK, BASE), jnp.float32),
    scratch_types=[
        pltpu.VMEM((_CH,), jnp.int32),
        pltpu.VMEM((_CH,), jnp.int32),
        pltpu.VMEM((_CH, BASE), jnp.float32),
        pltpu.SemaphoreType.DMA,
        pltpu.SemaphoreType.DMA,
    ],
    compiler_params=pltpu.CompilerParams(use_tc_tiling_on_sc=False),
)
def _sc_gather(c_hbm, idx_hbm, dst_hbm, out_hbm, idxc, dstc, rows, sem, sem2):
    wid = lax.axis_index("s") * _NC + lax.axis_index("c")
    base = wid * _PER_W
    for c in range(_NCHUNK):
        off = base + c * _CH
        pltpu.sync_copy(idx_hbm.at[pl.ds(off, _CH)], idxc)
        pltpu.sync_copy(dst_hbm.at[pl.ds(off, _CH)], dstc)
        pltpu.async_copy(c_hbm.at[idxc], rows, sem).wait()
        pltpu.async_copy(rows, out_hbm.at[dstc], sem2).wait()


_BC = 8192
_NBC = _B // _BC


def _tr_body(g_ref, out_ref):
    t = g_ref[...].T.reshape(4, BASE, _BC // 4)
    out_ref[...] = jnp.transpose(t, (1, 0, 2)).reshape(1, BASE, _BC)


def _transpose_out(g128):
    return pl.pallas_call(
        _tr_body,
        grid=(_L, _NBC),
        in_specs=[
            pl.BlockSpec((_BC // 4, 128), lambda l, b: (l * _NBC + b, 0)),
        ],
        out_specs=pl.BlockSpec((1, BASE, _BC), lambda l, b: (l, 0, b)),
        out_shape=jax.ShapeDtypeStruct((_L, BASE, _B), jnp.float32),
    )(g128)


def kernel(x, bucket_assignment, emb0, emb1, emb2, emb3, emb4, W3, b3, W4, b4):
    c_pad = _build_combined(bucket_assignment, emb0.T, emb1.T, emb2.T,
                            emb3.T, emb4.T, W3, b3, W4, b4)
    c_rows = c_pad.reshape(4 * V, BASE)
    idx = x.T.reshape(-1).astype(jnp.int32) * 4
    p = jnp.arange(_NTOK, dtype=jnp.int32)
    u, l_ = p % (_B), p // _B
    bc, v_ = u // _BC, u % _BC
    q, r = v_ // (_BC // 4), v_ % (_BC // 4)
    dst = ((l_ * _NBC + bc) * (_BC // 4) + r) * 4 + q
    g = _sc_gather(c_rows, idx, dst)
    g128 = g.reshape(_NTOK // 4, 128)
    out_phys = _transpose_out(g128)
    return jnp.transpose(out_phys, (2, 0, 1))


# speedup vs baseline: 1.9257x; 1.1382x over previous
import functools

import jax
import jax.numpy as jnp
from jax import lax
from jax.experimental import pallas as pl
from jax.experimental.pallas import tpu as pltpu
from jax.experimental.pallas import tpu_sc as plsc

V = 100000
BASE = 32
DIMS = (32, 32, 32, 51, 102)
_B, _L = 16384, 50


_CB = 8192
_GRID = (V + _CB - 1) // _CB


def _build_body(ba_ref, e0_ref, e1_ref, e2_ref, e3_ref, e4_ref,
                w3_ref, b3_ref, w4_ref, b4_ref, out_ref):
    ba = ba_ref[...]
    p3 = jnp.dot(w3_ref[...], e3_ref[...],
                 preferred_element_type=jnp.float32) + b3_ref[...].T
    p4 = jnp.dot(w4_ref[...], e4_ref[...],
                 preferred_element_type=jnp.float32) + b4_ref[...].T
    ct = jnp.where(ba == 0, e0_ref[...], 0.0)
    ct = ct + jnp.where(ba == 1, e1_ref[...], 0.0)
    ct = ct + jnp.where(ba == 2, e2_ref[...], 0.0)
    ct = ct + jnp.where(ba == 3, p3, 0.0)
    ct = ct + jnp.where(ba == 4, p4, 0.0)
    out_ref[:, :BASE] = ct.T


def _build_combined(ba, e0t, e1t, e2t, e3t, e4t, W3, b3, W4, b4):
    ba2 = ba.reshape(1, V).astype(jnp.int32)
    b3r = b3.reshape(1, BASE)
    b4r = b4.reshape(1, BASE)
    col = lambda i: (0, i)
    fixed = lambda i: (0, 0)
    return pl.pallas_call(
        _build_body,
        grid=(_GRID,),
        in_specs=[
            pl.BlockSpec((1, _CB), col),
            pl.BlockSpec((DIMS[0], _CB), col),
            pl.BlockSpec((DIMS[1], _CB), col),
            pl.BlockSpec((DIMS[2], _CB), col),
            pl.BlockSpec((DIMS[3], _CB), col),
            pl.BlockSpec((DIMS[4], _CB), col),
            pl.BlockSpec((BASE, DIMS[3]), fixed),
            pl.BlockSpec((1, BASE), fixed),
            pl.BlockSpec((BASE, DIMS[4]), fixed),
            pl.BlockSpec((1, BASE), fixed),
        ],
        out_specs=pl.BlockSpec((_CB, 128), lambda i: (i, 0)),
        out_shape=jax.ShapeDtypeStruct((V, 128), jnp.float32),
    )(ba2, e0t, e1t, e2t, e3t, e4t, W3, b3r, W4, b4r)


_NC = 2
_NS = 16
_NW = _NC * _NS
_NTOK = _B * _L
_PER_W = _NTOK // _NW
_CH = 2560
_NCHUNK = _PER_W // _CH


@functools.partial(
    pl.kernel,
    mesh=plsc.VectorSubcoreMesh(core_axis_name="c", subcore_axis_name="s",
                                num_cores=_NC),
    out_type=jax.ShapeDtypeStruct((_NTOK, BASE), jnp.float32),
    scratch_types=[
        pltpu.VMEM((_CH,), jnp.int32),
        pltpu.VMEM((_CH,), jnp.int32),
        pltpu.VMEM((_CH, BASE), jnp.float32),
        pltpu.SemaphoreType.DMA,
        pltpu.SemaphoreType.DMA,
    ],
    compiler_params=pltpu.CompilerParams(use_tc_tiling_on_sc=False),
)
def _sc_gather(c_hbm, idx_hbm, dst_hbm, out_hbm, idxc, dstc, rows, sem, sem2):
    wid = lax.axis_index("s") * _NC + lax.axis_index("c")
    base = wid * _PER_W
    for c in range(_NCHUNK):
        off = base + c * _CH
        pltpu.sync_copy(idx_hbm.at[pl.ds(off, _CH)], idxc)
        pltpu.sync_copy(dst_hbm.at[pl.ds(off, _CH)], dstc)
        pltpu.async_copy(c_hbm.at[idxc], rows, sem).wait()
        pltpu.async_copy(rows, out_hbm.at[dstc], sem2).wait()


_BC = 16384
_NBC = _B // _BC


def _tr_body(g_ref, out_ref):
    t = g_ref[...].T.reshape(4, BASE, _BC // 4)
    out_ref[...] = jnp.transpose(t, (1, 0, 2)).reshape(1, BASE, _BC)


def _transpose_out(g128):
    return pl.pallas_call(
        _tr_body,
        grid=(_L, _NBC),
        in_specs=[
            pl.BlockSpec((_BC // 4, 128), lambda l, b: (l * _NBC + b, 0)),
        ],
        out_specs=pl.BlockSpec((1, BASE, _BC), lambda l, b: (l, 0, b)),
        out_shape=jax.ShapeDtypeStruct((_L, BASE, _B), jnp.float32),
    )(g128)


def kernel(x, bucket_assignment, emb0, emb1, emb2, emb3, emb4, W3, b3, W4, b4):
    c_pad = _build_combined(bucket_assignment, emb0.T, emb1.T, emb2.T,
                            emb3.T, emb4.T, W3, b3, W4, b4)
    c_rows = c_pad.reshape(4 * V, BASE)
    idx = x.T.reshape(-1).astype(jnp.int32) * 4
    p = jnp.arange(_NTOK, dtype=jnp.int32)
    u, l_ = p % (_B), p // _B
    bc, v_ = u // _BC, u % _BC
    q, r = v_ // (_BC // 4), v_ % (_BC // 4)
    dst = ((l_ * _NBC + bc) * (_BC // 4) + r) * 4 + q
    g = _sc_gather(c_rows, idx, dst)
    g128 = g.reshape(_NTOK // 4, 128)
    out_phys = _transpose_out(g128)
    return jnp.transpose(out_phys, (2, 0, 1))
